# baseline (speedup 1.0000x reference)
import functools

import jax
import jax.numpy as jnp
from jax import lax
from jax.experimental import pallas as pl
from jax.experimental.pallas import tpu as pltpu, tpu_sc as plsc

NUM_SEEDS = 32
BATCH = 16384
IN_FEATURES = 7
NC, NS, L = 2, 16, 16
NW = NC * NS
B_PER_W = BATCH // NW
HALF = B_PER_W // 2
HGROUPS = HALF // L


def _sc_body(
    sdow_hbm, idx_hbm, w_hbm, out_hbm,
    idx0_v, idx1_v, day0_v, day1_v, w_v, out0_v, out1_v,
    sem_g0, sem_g1, sem_w0, sem_w1,
):
    wid = lax.axis_index("s") * NC + lax.axis_index("c")
    base = wid * B_PER_W

    pltpu.sync_copy(idx_hbm.at[pl.ds(base, HALF)], idx0_v)
    g0 = pltpu.async_copy(sdow_hbm.at[idx0_v], day0_v, sem_g0)
    pltpu.sync_copy(idx_hbm.at[pl.ds(base + HALF, HALF)], idx1_v)
    g1 = pltpu.async_copy(sdow_hbm.at[idx1_v], day1_v, sem_g1)
    pltpu.sync_copy(w_hbm, w_v)

    def compute_half(day_v, out_v):
        @plsc.parallel_loop(0, HGROUPS)
        def group(g):
            day_vec = day_v[pl.ds(g * L, L)]
            for s in range(NUM_SEEDS):
                out_v[s, pl.ds(g * L, L)] = plsc.load_gather(
                    w_v.at[s], [day_vec]
                )

    g0.wait()
    compute_half(day0_v, out0_v)
    w0 = pltpu.async_copy(
        out0_v, out_hbm.at[:, pl.ds(base, HALF)], sem_w0
    )
    g1.wait()
    compute_half(day1_v, out1_v)
    w1 = pltpu.async_copy(
        out1_v, out_hbm.at[:, pl.ds(base + HALF, HALF)], sem_w1
    )
    w0.wait()
    w1.wait()


@jax.jit
def kernel(session_day_of_week, session_index, W):
    mesh = plsc.VectorSubcoreMesh(
        core_axis_name="c", subcore_axis_name="s", num_cores=NC, num_subcores=NS
    )
    run = functools.partial(
        pl.kernel,
        out_type=jax.ShapeDtypeStruct((NUM_SEEDS, BATCH), jnp.float32),
        mesh=mesh,
        scratch_types=[
            pltpu.VMEM((HALF,), jnp.int32),
            pltpu.VMEM((HALF,), jnp.int32),
            pltpu.VMEM((HALF,), jnp.int32),
            pltpu.VMEM((HALF,), jnp.int32),
            pltpu.VMEM((NUM_SEEDS, 8), jnp.float32),
            pltpu.VMEM((NUM_SEEDS, HALF), jnp.float32),
            pltpu.VMEM((NUM_SEEDS, HALF), jnp.float32),
            pltpu.SemaphoreType.DMA,
            pltpu.SemaphoreType.DMA,
            pltpu.SemaphoreType.DMA,
            pltpu.SemaphoreType.DMA,
        ],
        compiler_params=pltpu.CompilerParams(needs_layout_passes=False),
    )(_sc_body)
    return run(
        session_day_of_week.astype(jnp.int32),
        session_index.astype(jnp.int32),
        jnp.pad(W, ((0, 0), (0, 8 - IN_FEATURES))),
    )


# speedup vs baseline: 1.0721x; 1.0721x over previous
import functools

import jax
import jax.numpy as jnp
from jax import lax
from jax.experimental import pallas as pl
from jax.experimental.pallas import tpu as pltpu, tpu_sc as plsc

NUM_SEEDS = 32
BATCH = 16384
IN_FEATURES = 7
NC, NS, L = 2, 16, 16
NW = NC * NS
B_PER_W = BATCH // NW
GROUPS = B_PER_W // L


def _sc_body(
    sdow_hbm, idx_hbm, w_hbm, out_hbm, idx_v, day_v, w_v, out_v,
    sem_i, sem_w, sem_g,
):
    wid = lax.axis_index("s") * NC + lax.axis_index("c")
    base = wid * B_PER_W

    ci = pltpu.async_copy(idx_hbm.at[pl.ds(base, B_PER_W)], idx_v, sem_i)
    cw = pltpu.async_copy(w_hbm, w_v, sem_w)
    ci.wait()
    cg = pltpu.async_copy(sdow_hbm.at[idx_v], day_v, sem_g)
    cw.wait()
    cg.wait()

    @plsc.parallel_loop(0, GROUPS)
    def group(g):
        day_vec = day_v[pl.ds(g * L, L)]
        for s in range(NUM_SEEDS):
            out_v[s, pl.ds(g * L, L)] = plsc.load_gather(w_v.at[s], [day_vec])

    pltpu.sync_copy(out_v, out_hbm.at[:, pl.ds(base, B_PER_W)])


@jax.jit
def kernel(session_day_of_week, session_index, W):
    mesh = plsc.VectorSubcoreMesh(
        core_axis_name="c", subcore_axis_name="s", num_cores=NC, num_subcores=NS
    )
    run = functools.partial(
        pl.kernel,
        out_type=jax.ShapeDtypeStruct((NUM_SEEDS, BATCH), jnp.float32),
        mesh=mesh,
        scratch_types=[
            pltpu.VMEM((B_PER_W,), jnp.int32),
            pltpu.VMEM((B_PER_W,), jnp.int32),
            pltpu.VMEM((NUM_SEEDS, IN_FEATURES), jnp.float32),
            pltpu.VMEM((NUM_SEEDS, B_PER_W), jnp.float32),
            pltpu.SemaphoreType.DMA,
            pltpu.SemaphoreType.DMA,
            pltpu.SemaphoreType.DMA,
        ],
        compiler_params=pltpu.CompilerParams(needs_layout_passes=False),
    )(_sc_body)
    return run(
        session_day_of_week.astype(jnp.int32),
        session_index.astype(jnp.int32),
        W,
    )
